# baseline (speedup 1.0000x reference)
import functools

import jax
import jax.numpy as jnp
from jax import lax
from jax.experimental import pallas as pl
from jax.experimental.pallas import tpu as pltpu
from jax.experimental.pallas import tpu_sc as plsc

_L = 16


@functools.lru_cache(maxsize=None)
def _make_sc_kernel(n, n_rows, chunk):
    info = plsc.get_sparse_core_info()
    nc, ns = info.num_cores, info.num_subcores
    nw = nc * ns
    assert n % nw == 0
    per_w = n // nw
    assert per_w % chunk == 0 and chunk % _L == 0
    n_chunks = per_w // chunk
    assert n_chunks % 2 == 0 and n_chunks >= 4

    mesh = plsc.VectorSubcoreMesh(core_axis_name="c", subcore_axis_name="s")

    @functools.partial(
        pl.kernel,
        out_type=jax.ShapeDtypeStruct((3, n), jnp.float32),
        mesh=mesh,
        scratch_types=[
            [pltpu.VMEM((chunk,), jnp.int32) for _ in range(2)],
            [pltpu.VMEM((chunk, _L), jnp.float32) for _ in range(2)],
            pltpu.VMEM((3, chunk), jnp.float32),
            pltpu.VMEM((3, chunk), jnp.float32),
            [pltpu.SemaphoreType.DMA for _ in range(3)],
        ],
        compiler_params=pltpu.CompilerParams(
            needs_layout_passes=False, use_tc_tiling_on_sc=False
        ),
    )
    def sc_kernel(pos_hbm, idx_hbm, table_hbm, out_hbm,
                  idx_v, m_v, pos_v, out_v, sems):
        wid = lax.axis_index("s") * nc + lax.axis_index("c")
        lanes = lax.iota(jnp.int32, _L)
        k_splats = [jnp.full((_L,), k, jnp.int32) for k in range(12)]

        def base_of(i):
            return wid * per_w + i * chunk

        def gather_desc(p):
            return pltpu.make_async_copy(
                table_hbm.at[idx_v[p]], m_v[p], sems[p])

        def prefetch(i, p):
            pltpu.sync_copy(idx_hbm.at[pl.ds(base_of(i), chunk)], idx_v[p])
            gather_desc(p).start()

        def compute(p):
            mr = m_v[p]

            def group_body(gi, c2):
                g0 = gi * _L
                e = g0 + lanes
                x = pos_v[0, pl.ds(g0, _L)]
                y = pos_v[1, pl.ds(g0, _L)]
                z = pos_v[2, pl.ds(g0, _L)]
                m = [plsc.load_gather(mr, [e, k_splats[k]])
                     for k in range(12)]
                out_v[0, pl.ds(g0, _L)] = m[0] * x + m[1] * y + m[2] * z + m[3]
                out_v[1, pl.ds(g0, _L)] = m[4] * x + m[5] * y + m[6] * z + m[7]
                out_v[2, pl.ds(g0, _L)] = (
                    m[8] * x + m[9] * y + m[10] * z + m[11])
                return c2

            lax.fori_loop(0, chunk // _L, group_body, 0, unroll=8)

        def step(i, p, tail):
            if not tail:
                prefetch(i + 1, 1 - p)
            pd = pltpu.async_copy(
                pos_hbm.at[:, pl.ds(base_of(i), chunk)], pos_v, sems[2])
            gather_desc(p).wait()
            pd.wait()
            compute(p)
            pltpu.sync_copy(out_v, out_hbm.at[:, pl.ds(base_of(i), chunk)])

        prefetch(0, 0)

        def loop_body(j, carry):
            step(2 * j, 0, False)
            step(2 * j + 1, 1, False)
            return carry

        lax.fori_loop(0, n_chunks // 2 - 1, loop_body, 0)
        step(n_chunks - 2, 0, False)
        step(n_chunks - 1, 1, True)

    return sc_kernel


def kernel(pos_3d, portal_idx, transform):
    b, s, _ = pos_3d.shape
    n = b * s
    p = transform.shape[0]
    pos = jnp.transpose(pos_3d, (2, 1, 0)).reshape(3, n)
    idx = jnp.transpose(portal_idx, (1, 0)).reshape(n).astype(jnp.int32)
    table = jnp.pad(transform.reshape(p, 12), ((0, 0), (0, 4)))
    out = _make_sc_kernel(n, p, 2560)(pos, idx, table)
    return jnp.transpose(out.reshape(3, s, b), (2, 1, 0))


# speedup vs baseline: 1.0222x; 1.0222x over previous
import functools

import jax
import jax.numpy as jnp
from jax import lax
from jax.experimental import pallas as pl
from jax.experimental.pallas import tpu as pltpu
from jax.experimental.pallas import tpu_sc as plsc

_L = 16


@functools.lru_cache(maxsize=None)
def _make_sc_kernel(n, n_rows, chunk):
    info = plsc.get_sparse_core_info()
    nc, ns = info.num_cores, info.num_subcores
    nw = nc * ns
    assert n % nw == 0
    per_w = n // nw
    assert per_w % chunk == 0 and chunk % _L == 0
    n_chunks = per_w // chunk
    assert n_chunks % 2 == 0 and n_chunks >= 4

    mesh = plsc.VectorSubcoreMesh(core_axis_name="c", subcore_axis_name="s")

    @functools.partial(
        pl.kernel,
        out_type=jax.ShapeDtypeStruct((3, n), jnp.float32),
        mesh=mesh,
        scratch_types=[
            [pltpu.VMEM((chunk,), jnp.int32) for _ in range(2)],
            [pltpu.VMEM((chunk, _L), jnp.float32) for _ in range(2)],
            pltpu.VMEM((3, chunk), jnp.float32),
            [pltpu.VMEM((3, chunk), jnp.float32) for _ in range(2)],
            [pltpu.SemaphoreType.DMA for _ in range(5)],
        ],
        compiler_params=pltpu.CompilerParams(
            needs_layout_passes=False, use_tc_tiling_on_sc=False
        ),
    )
    def sc_kernel(pos_hbm, idx_hbm, table_hbm, out_hbm,
                  idx_v, m_v, pos_v, out_v, sems):
        wid = lax.axis_index("s") * nc + lax.axis_index("c")
        lanes = lax.iota(jnp.int32, _L)
        k_splats = [jnp.full((_L,), k, jnp.int32) for k in range(12)]

        def base_of(i):
            return wid * per_w + i * chunk

        def gather_desc(p):
            return pltpu.make_async_copy(
                table_hbm.at[idx_v[p]], m_v[p], sems[p])

        def prefetch(i, p):
            pltpu.sync_copy(idx_hbm.at[pl.ds(base_of(i), chunk)], idx_v[p])
            gather_desc(p).start()

        def out_desc(i, p):
            return pltpu.make_async_copy(
                out_v[p], out_hbm.at[:, pl.ds(base_of(i), chunk)],
                sems[3 + p])

        def compute(p):
            mr = m_v[p]

            def group_body(gi, c2):
                g0 = gi * _L
                e = g0 + lanes
                x = pos_v[0, pl.ds(g0, _L)]
                y = pos_v[1, pl.ds(g0, _L)]
                z = pos_v[2, pl.ds(g0, _L)]
                m = [plsc.load_gather(mr, [e, k_splats[k]])
                     for k in range(12)]
                ov = out_v[p]
                ov[0, pl.ds(g0, _L)] = m[0] * x + m[1] * y + m[2] * z + m[3]
                ov[1, pl.ds(g0, _L)] = m[4] * x + m[5] * y + m[6] * z + m[7]
                ov[2, pl.ds(g0, _L)] = m[8] * x + m[9] * y + m[10] * z + m[11]
                return c2

            lax.fori_loop(0, chunk // _L, group_body, 0, unroll=8)

        def step(i, p, head, tail):
            if not tail:
                prefetch(i + 1, 1 - p)
            pd = pltpu.async_copy(
                pos_hbm.at[:, pl.ds(base_of(i), chunk)], pos_v, sems[2])
            gather_desc(p).wait()
            pd.wait()
            if not head:
                out_desc(i - 2, p).wait()
            compute(p)
            out_desc(i, p).start()

        prefetch(0, 0)
        step(0, 0, True, False)
        step(1, 1, True, False)

        def loop_body(j, carry):
            step(2 * j, 0, False, False)
            step(2 * j + 1, 1, False, False)
            return carry

        lax.fori_loop(1, n_chunks // 2 - 1, loop_body, 0)
        step(n_chunks - 2, 0, False, False)
        step(n_chunks - 1, 1, False, True)
        out_desc(n_chunks - 2, 0).wait()
        out_desc(n_chunks - 1, 1).wait()

    return sc_kernel


def kernel(pos_3d, portal_idx, transform):
    b, s, _ = pos_3d.shape
    n = b * s
    p = transform.shape[0]
    pos = jnp.transpose(pos_3d, (2, 1, 0)).reshape(3, n)
    idx = jnp.transpose(portal_idx, (1, 0)).reshape(n).astype(jnp.int32)
    table = jnp.pad(transform.reshape(p, 12), ((0, 0), (0, 4)))
    out = _make_sc_kernel(n, p, 2560)(pos, idx, table)
    return jnp.transpose(out.reshape(3, s, b), (2, 1, 0))


# speedup vs baseline: 1.2047x; 1.1785x over previous
import functools

import jax
import jax.numpy as jnp
from jax import lax
from jax.experimental import pallas as pl
from jax.experimental.pallas import tpu as pltpu
from jax.experimental.pallas import tpu_sc as plsc

_L = 16


@functools.lru_cache(maxsize=None)
def _make_sc_kernel(n, n_rows, chunk):
    info = plsc.get_sparse_core_info()
    nc, ns = info.num_cores, info.num_subcores
    nw = nc * ns
    assert n % nw == 0
    per_w = n // nw
    assert per_w % chunk == 0 and chunk % _L == 0
    n_chunks = per_w // chunk
    assert n_chunks % 2 == 0 and n_chunks >= 4

    mesh = plsc.VectorSubcoreMesh(core_axis_name="c", subcore_axis_name="s")

    @functools.partial(
        pl.kernel,
        out_type=jax.ShapeDtypeStruct((3, n), jnp.float32),
        mesh=mesh,
        scratch_types=[
            [pltpu.VMEM((chunk,), jnp.int32) for _ in range(2)],
            [pltpu.VMEM((chunk, _L), jnp.float32) for _ in range(2)],
            pltpu.VMEM((3, chunk), jnp.float32),
            [pltpu.VMEM((3, chunk), jnp.float32) for _ in range(2)],
            [pltpu.SemaphoreType.DMA for _ in range(5)],
        ],
        compiler_params=pltpu.CompilerParams(
            needs_layout_passes=False, use_tc_tiling_on_sc=False
        ),
    )
    def sc_kernel(pos_hbm, idx_hbm, table_hbm, out_hbm,
                  idx_v, m_v, pos_v, out_v, sems):
        wid = lax.axis_index("s") * nc + lax.axis_index("c")
        lanes = lax.iota(jnp.int32, _L)
        k_splats = [jnp.full((_L,), k, jnp.int32) for k in range(12)]

        def base_of(i):
            return wid * per_w + i * chunk

        def gather_desc(p):
            return pltpu.make_async_copy(
                table_hbm.at[idx_v[p]], m_v[p], sems[p])

        def prefetch(i, p):
            pltpu.sync_copy(idx_hbm.at[pl.ds(base_of(i), chunk)], idx_v[p])
            gather_desc(p).start()

        def out_desc(i, p):
            return pltpu.make_async_copy(
                out_v[p], out_hbm.at[:, pl.ds(base_of(i), chunk)],
                sems[3 + p])

        def compute(p):
            mr = m_v[p]

            def group_body(gi, c2):
                g0 = gi * _L
                e = g0 + lanes
                x = pos_v[0, pl.ds(g0, _L)]
                y = pos_v[1, pl.ds(g0, _L)]
                z = pos_v[2, pl.ds(g0, _L)]
                m = [plsc.load_gather(mr, [e, k_splats[k]])
                     for k in range(12)]
                ov = out_v[p]
                ov[0, pl.ds(g0, _L)] = m[0] * x + m[1] * y + m[2] * z + m[3]
                ov[1, pl.ds(g0, _L)] = m[4] * x + m[5] * y + m[6] * z + m[7]
                ov[2, pl.ds(g0, _L)] = m[8] * x + m[9] * y + m[10] * z + m[11]
                return c2

            lax.fori_loop(0, chunk // _L, group_body, 0, unroll=8)

        def step(i, p, head, tail):
            if not tail:
                prefetch(i + 1, 1 - p)
            pd = pltpu.async_copy(
                pos_hbm.at[:, pl.ds(base_of(i), chunk)], pos_v, sems[2])
            gather_desc(p).wait()
            pd.wait()
            if not head:
                out_desc(i - 2, p).wait()
            compute(p)
            out_desc(i, p).start()

        prefetch(0, 0)
        step(0, 0, True, False)
        step(1, 1, True, False)

        def loop_body(j, carry):
            step(2 * j, 0, False, False)
            step(2 * j + 1, 1, False, False)
            return carry

        lax.fori_loop(1, n_chunks // 2 - 1, loop_body, 0)
        step(n_chunks - 2, 0, False, False)
        step(n_chunks - 1, 1, False, True)
        out_desc(n_chunks - 2, 0).wait()
        out_desc(n_chunks - 1, 1).wait()

    return sc_kernel


def kernel(pos_3d, portal_idx, transform):
    b, s, _ = pos_3d.shape
    n = b * s
    p = transform.shape[0]
    st, sb = s // 8, b // 128
    pos = (jnp.transpose(pos_3d, (2, 1, 0))
           .reshape(3, st, 8, sb, 128)
           .transpose(0, 1, 3, 2, 4)
           .reshape(3, n))
    idx = (jnp.transpose(portal_idx, (1, 0))
           .reshape(st, 8, sb, 128)
           .transpose(0, 2, 1, 3)
           .reshape(n)
           .astype(jnp.int32))
    table = jnp.pad(transform.reshape(p, 12), ((0, 0), (0, 4)))
    out = _make_sc_kernel(n, p, 2560)(pos, idx, table)
    out = (out.reshape(3, st, sb, 8, 128)
           .transpose(0, 1, 3, 2, 4)
           .reshape(3, s, b))
    return jnp.transpose(out, (2, 1, 0))


# speedup vs baseline: 1.2914x; 1.0719x over previous
import functools

import jax
import jax.numpy as jnp
from jax import lax
from jax.experimental import pallas as pl
from jax.experimental.pallas import tpu as pltpu
from jax.experimental.pallas import tpu_sc as plsc

_L = 16


@functools.lru_cache(maxsize=None)
def _make_sc_kernel(n, n_rows, chunk):
    info = plsc.get_sparse_core_info()
    nc, ns = info.num_cores, info.num_subcores
    nw = nc * ns
    assert n % nw == 0
    per_w = n // nw
    assert per_w % chunk == 0 and chunk % _L == 0
    n_chunks = per_w // chunk
    assert n_chunks % 2 == 0 and n_chunks >= 4

    mesh = plsc.VectorSubcoreMesh(core_axis_name="c", subcore_axis_name="s")

    @functools.partial(
        pl.kernel,
        out_type=jax.ShapeDtypeStruct((3, n), jnp.float32),
        mesh=mesh,
        scratch_types=[
            [pltpu.VMEM((chunk,), jnp.int32) for _ in range(2)],
            [pltpu.VMEM((chunk, _L), jnp.float32) for _ in range(2)],
            pltpu.VMEM((3, chunk), jnp.float32),
            [pltpu.VMEM((3, chunk), jnp.float32) for _ in range(2)],
            [pltpu.SemaphoreType.DMA for _ in range(7)],
        ],
        compiler_params=pltpu.CompilerParams(
            needs_layout_passes=False, use_tc_tiling_on_sc=False
        ),
    )
    def sc_kernel(pos_hbm, idx_hbm, table_hbm, out_hbm,
                  idx_v, m_v, pos_v, out_v, sems):
        wid = lax.axis_index("s") * nc + lax.axis_index("c")
        lanes = lax.iota(jnp.int32, _L)
        k_splats = [jnp.full((_L,), k, jnp.int32) for k in range(12)]

        def base_of(i):
            return wid * per_w + i * chunk

        def gather_desc(p):
            return pltpu.make_async_copy(
                table_hbm.at[idx_v[p]], m_v[p], sems[p])

        def idx_desc(i, p):
            return pltpu.make_async_copy(
                idx_hbm.at[pl.ds(base_of(i), chunk)], idx_v[p], sems[5 + p])

        def out_desc(i, p):
            return pltpu.make_async_copy(
                out_v[p], out_hbm.at[:, pl.ds(base_of(i), chunk)],
                sems[3 + p])

        def compute(p):
            mr = m_v[p]

            def group_body(gi, c2):
                g0 = gi * _L
                e = g0 + lanes
                x = pos_v[0, pl.ds(g0, _L)]
                y = pos_v[1, pl.ds(g0, _L)]
                z = pos_v[2, pl.ds(g0, _L)]
                m = [plsc.load_gather(mr, [e, k_splats[k]])
                     for k in range(12)]
                ov = out_v[p]
                ov[0, pl.ds(g0, _L)] = m[0] * x + m[1] * y + m[2] * z + m[3]
                ov[1, pl.ds(g0, _L)] = m[4] * x + m[5] * y + m[6] * z + m[7]
                ov[2, pl.ds(g0, _L)] = m[8] * x + m[9] * y + m[10] * z + m[11]
                return c2

            lax.fori_loop(0, chunk // _L, group_body, 0, unroll=16)

        def step(i, p, head, tail1, tail2):
            q = 1 - p
            if not tail1:
                idx_desc(i + 1, q).wait()
                gather_desc(q).start()
            pd = pltpu.async_copy(
                pos_hbm.at[:, pl.ds(base_of(i), chunk)], pos_v, sems[2])
            gather_desc(p).wait()
            if not tail2:
                idx_desc(i + 2, p).start()
            pd.wait()
            if not head:
                out_desc(i - 2, p).wait()
            compute(p)
            out_desc(i, p).start()

        idx_desc(0, 0).start()
        idx_desc(0, 0).wait()
        gather_desc(0).start()
        idx_desc(1, 1).start()

        step(0, 0, True, False, False)
        step(1, 1, True, False, False)

        def loop_body(j, carry):
            step(2 * j, 0, False, False, False)
            step(2 * j + 1, 1, False, False, False)
            return carry

        lax.fori_loop(1, n_chunks // 2 - 1, loop_body, 0)
        step(n_chunks - 2, 0, False, False, True)
        step(n_chunks - 1, 1, False, True, True)
        out_desc(n_chunks - 2, 0).wait()
        out_desc(n_chunks - 1, 1).wait()

    return sc_kernel


def kernel(pos_3d, portal_idx, transform):
    b, s, _ = pos_3d.shape
    n = b * s
    p = transform.shape[0]
    st, sb = s // 8, b // 128
    pos = (jnp.transpose(pos_3d, (2, 1, 0))
           .reshape(3, st, 8, sb, 128)
           .transpose(0, 1, 3, 2, 4)
           .reshape(3, n))
    idx = (jnp.transpose(portal_idx, (1, 0))
           .reshape(st, 8, sb, 128)
           .transpose(0, 2, 1, 3)
           .reshape(n)
           .astype(jnp.int32))
    table = jnp.pad(transform.reshape(p, 12), ((0, 0), (0, 4)))
    out = _make_sc_kernel(n, p, 2560)(pos, idx, table)
    out = (out.reshape(3, st, sb, 8, 128)
           .transpose(0, 1, 3, 2, 4)
           .reshape(3, s, b))
    return jnp.transpose(out, (2, 1, 0))


# speedup vs baseline: 1.4978x; 1.1598x over previous
import functools

import jax
import jax.numpy as jnp
from jax import lax
from jax.experimental import pallas as pl
from jax.experimental.pallas import tpu as pltpu
from jax.experimental.pallas import tpu_sc as plsc

_L = 16


@functools.lru_cache(maxsize=None)
def _make_sc_kernel(n, n_rows, chunk):
    info = plsc.get_sparse_core_info()
    nc, ns = info.num_cores, info.num_subcores
    nw = nc * ns
    assert n % nw == 0
    per_w = n // nw
    assert per_w % chunk == 0 and chunk % _L == 0
    n_chunks = per_w // chunk
    assert n_chunks % 2 == 0 and n_chunks >= 4

    mesh = plsc.VectorSubcoreMesh(core_axis_name="c", subcore_axis_name="s")

    @functools.partial(
        pl.kernel,
        out_type=jax.ShapeDtypeStruct((3, n), jnp.float32),
        mesh=mesh,
        scratch_types=[
            [pltpu.VMEM((chunk,), jnp.int32) for _ in range(2)],
            [pltpu.VMEM((chunk, _L), jnp.float32) for _ in range(2)],
            [pltpu.VMEM((3, chunk), jnp.float32) for _ in range(2)],
            [pltpu.VMEM((3, chunk), jnp.float32) for _ in range(2)],
            [pltpu.SemaphoreType.DMA for _ in range(9)],
        ],
        compiler_params=pltpu.CompilerParams(
            needs_layout_passes=False, use_tc_tiling_on_sc=False
        ),
    )
    def sc_kernel(pos_hbm, idx_hbm, table_hbm, out_hbm,
                  idx_v, m_v, pos_v, out_v, sems):
        wid = lax.axis_index("s") * nc + lax.axis_index("c")
        lanes = lax.iota(jnp.int32, _L)
        k_splats = [jnp.full((_L,), k, jnp.int32) for k in range(12)]

        def base_of(i):
            return wid * per_w + i * chunk

        def gather_desc(p):
            return pltpu.make_async_copy(
                table_hbm.at[idx_v[p]], m_v[p], sems[p])

        def idx_desc(i, p):
            return pltpu.make_async_copy(
                idx_hbm.at[pl.ds(base_of(i), chunk)], idx_v[p], sems[5 + p])

        def pos_desc(i, p):
            return pltpu.make_async_copy(
                pos_hbm.at[:, pl.ds(base_of(i), chunk)], pos_v[p],
                sems[7 + p])

        def out_desc(i, p):
            return pltpu.make_async_copy(
                out_v[p], out_hbm.at[:, pl.ds(base_of(i), chunk)],
                sems[3 + p])

        def compute(p):
            mr = m_v[p]

            def group_body(gi, c2):
                g0 = gi * _L
                e = g0 + lanes
                x = pos_v[p][0, pl.ds(g0, _L)]
                y = pos_v[p][1, pl.ds(g0, _L)]
                z = pos_v[p][2, pl.ds(g0, _L)]
                m = [plsc.load_gather(mr, [e, k_splats[k]])
                     for k in range(12)]
                ov = out_v[p]
                ov[0, pl.ds(g0, _L)] = m[0] * x + m[1] * y + m[2] * z + m[3]
                ov[1, pl.ds(g0, _L)] = m[4] * x + m[5] * y + m[6] * z + m[7]
                ov[2, pl.ds(g0, _L)] = m[8] * x + m[9] * y + m[10] * z + m[11]
                return c2

            lax.fori_loop(0, chunk // _L, group_body, 0, unroll=16)

        def step(i, p, head, tail1, tail2):
            q = 1 - p
            if not tail1:
                idx_desc(i + 1, q).wait()
                gather_desc(q).start()
                pos_desc(i + 1, q).start()
            gather_desc(p).wait()
            if not tail2:
                idx_desc(i + 2, p).start()
            pos_desc(i, p).wait()
            if not head:
                out_desc(i - 2, p).wait()
            compute(p)
            out_desc(i, p).start()

        idx_desc(0, 0).start()
        idx_desc(0, 0).wait()
        gather_desc(0).start()
        pos_desc(0, 0).start()
        idx_desc(1, 1).start()

        step(0, 0, True, False, False)
        step(1, 1, True, False, False)

        def loop_body(j, carry):
            step(2 * j, 0, False, False, False)
            step(2 * j + 1, 1, False, False, False)
            return carry

        lax.fori_loop(1, n_chunks // 2 - 1, loop_body, 0)
        step(n_chunks - 2, 0, False, False, True)
        step(n_chunks - 1, 1, False, True, True)
        out_desc(n_chunks - 2, 0).wait()
        out_desc(n_chunks - 1, 1).wait()

    return sc_kernel


def kernel(pos_3d, portal_idx, transform):
    b, s, _ = pos_3d.shape
    n = b * s
    p = transform.shape[0]
    st, sb = s // 8, b // 128
    pos = (jnp.transpose(pos_3d, (2, 1, 0))
           .reshape(3, st, 8, sb, 128)
           .transpose(0, 1, 3, 2, 4)
           .reshape(3, n))
    idx = (jnp.transpose(portal_idx, (1, 0))
           .reshape(st, 8, sb, 128)
           .transpose(0, 2, 1, 3)
           .reshape(n)
           .astype(jnp.int32))
    table = jnp.pad(transform.reshape(p, 12), ((0, 0), (0, 4)))
    out = _make_sc_kernel(n, p, 2048)(pos, idx, table)
    out = (out.reshape(3, st, sb, 8, 128)
           .transpose(0, 1, 3, 2, 4)
           .reshape(3, s, b))
    return jnp.transpose(out, (2, 1, 0))
